# Initial kernel scaffold; baseline (speedup 1.0000x reference)
import jax
import jax.numpy as jnp
from jax.experimental import pallas as pl


def kernel(features, coordinates, p):
    raise NotImplementedError("write your pallas kernel here")


# speedup vs baseline: 8.8188x; 8.8188x over previous
import functools

import jax
import jax.numpy as jnp
from jax import lax
from jax.experimental import pallas as pl
from jax.experimental.pallas import tpu as pltpu

N = 32768
D = 512
B = 16
EPS = 1e-06

_BLK = 1024
_GRID = N // _BLK


def _tc_body(ids_ref, x_ref, p_ref, out_ref, cnt_ref):
    i = pl.program_id(0)

    @pl.when(i == 0)
    def _init():
        out_ref[...] = jnp.zeros_like(out_ref)
        cnt_ref[...] = jnp.zeros_like(cnt_ref)

    x = x_ref[...]
    p = p_ref[0]
    powered = jnp.exp(p * jnp.log(jnp.maximum(x, EPS)))

    ids = ids_ref[0, 0, :]
    onehot = (ids[:, None] == lax.broadcasted_iota(jnp.int32, (_BLK, B), 1)
              ).astype(jnp.float32)
    partial = lax.dot_general(
        onehot, powered, (((0,), (0,)), ((), ())),
        preferred_element_type=jnp.float32)
    out_ref[...] += partial
    cnt_ref[...] += jnp.broadcast_to(jnp.sum(onehot, axis=0)[:, None], (B, 128))

    @pl.when(i == _GRID - 1)
    def _finalize():
        sums = out_ref[...]
        counts = cnt_ref[...][:, 0:1]
        mean = sums / jnp.maximum(counts, 1.0)
        out_ref[...] = jnp.exp(jnp.log(mean) / p)


@functools.partial(jax.jit, static_argnames=("interpret",))
def _tc_gem(features, ids3d, p, interpret=False):
    return pl.pallas_call(
        _tc_body,
        out_shape=jax.ShapeDtypeStruct((B, D), jnp.float32),
        grid=(_GRID,),
        in_specs=[
            pl.BlockSpec((1, 1, _BLK), lambda i: (i, 0, 0)),
            pl.BlockSpec((_BLK, D), lambda i: (i, 0)),
            pl.BlockSpec(memory_space=pltpu.SMEM),
        ],
        out_specs=pl.BlockSpec((B, D), lambda i: (0, 0)),
        scratch_shapes=[pltpu.VMEM((B, 128), jnp.float32)],
        compiler_params=pltpu.CompilerParams(
            dimension_semantics=("arbitrary",)),
        interpret=interpret,
    )(ids3d, features, p)


def kernel(features, coordinates, p):
    ids3d = coordinates[:, 0].astype(jnp.int32).reshape(_GRID, 1, _BLK)
    return _tc_gem(features, ids3d, p)
